# Initial kernel scaffold; baseline (speedup 1.0000x reference)
import jax
import jax.numpy as jnp
from jax.experimental import pallas as pl


def kernel(x, tables):
    raise NotImplementedError("write your pallas kernel here")


# speedup vs baseline: 1.1370x; 1.1370x over previous
import functools

import jax
import jax.numpy as jnp
from jax import lax
from jax.experimental import pallas as pl
from jax.experimental.pallas import tpu as pltpu
from jax.experimental.pallas import tpu_sc as plsc

NUM_FIELDS = 26
VOCAB = 100000
EMB_DIM = 16
BATCH = 16384

NC = 2
NS = 16
LANES = 16
NW = NC * NS
ROWS_PER_W = BATCH // NW
P_ROWS = 64
N_PASS = ROWS_PER_W // P_ROWS
IDX_PER_PASS = P_ROWS * NUM_FIELDS
SUB = 128
N_SUB = IDX_PER_PASS // SUB


def _build_sc_gather():
    mesh = plsc.VectorSubcoreMesh(core_axis_name="c", subcore_axis_name="s")

    @functools.partial(
        pl.kernel,
        mesh=mesh,
        compiler_params=pltpu.CompilerParams(use_tc_tiling_on_sc=False),
        out_type=jax.ShapeDtypeStruct((BATCH * NUM_FIELDS, EMB_DIM),
                                      jnp.float32),
        scratch_types=[
            pltpu.VMEM((IDX_PER_PASS,), jnp.int32),
            pltpu.VMEM((IDX_PER_PASS,), jnp.int32),
            pltpu.VMEM((IDX_PER_PASS, EMB_DIM), jnp.float32),
            pltpu.SemaphoreType.DMA,
        ],
    )
    def k(x_hbm, offs_hbm, tab_hbm, out_hbm, offs_v, idx_v, rows_v, sem):
        wid = lax.axis_index("s") * NC + lax.axis_index("c")
        pltpu.sync_copy(offs_hbm, offs_v)

        def one_pass(p, carry):
            gbase = (wid * N_PASS + p) * IDX_PER_PASS
            pltpu.sync_copy(x_hbm.at[pl.ds(gbase, IDX_PER_PASS)], idx_v)

            def add_offs(j, c):
                s = pl.ds(j * LANES, LANES)
                idx_v[s] = idx_v[s] + offs_v[s]
                return c

            lax.fori_loop(0, IDX_PER_PASS // LANES, add_offs, 0)

            copies = []
            for j in range(N_SUB):
                copies.append(pltpu.async_copy(
                    tab_hbm.at[idx_v.at[pl.ds(j * SUB, SUB)]],
                    rows_v.at[pl.ds(j * SUB, SUB)],
                    sem))
            for c in copies:
                c.wait()

            pltpu.sync_copy(rows_v, out_hbm.at[pl.ds(gbase, IDX_PER_PASS)])
            return carry

        lax.fori_loop(0, N_PASS, one_pass, 0)

    return k


def kernel(x, tables):
    x_flat = x.reshape(BATCH * NUM_FIELDS)
    tab_flat = tables.reshape(NUM_FIELDS * VOCAB, EMB_DIM)
    offs = jnp.tile(jnp.arange(NUM_FIELDS, dtype=jnp.int32) * VOCAB, P_ROWS)
    out = _build_sc_gather()(x_flat, offs, tab_flat)
    return out.reshape(BATCH, NUM_FIELDS * EMB_DIM)


# speedup vs baseline: 1.1436x; 1.0058x over previous
import functools

import jax
import jax.numpy as jnp
from jax import lax
from jax.experimental import pallas as pl
from jax.experimental.pallas import tpu as pltpu
from jax.experimental.pallas import tpu_sc as plsc

NUM_FIELDS = 26
VOCAB = 100000
EMB_DIM = 16
BATCH = 16384

NC = 2
NS = 16
LANES = 16
NW = NC * NS
ROWS_PER_W = BATCH // NW
P_ROWS = 128
N_PASS = ROWS_PER_W // P_ROWS
N_VEC = P_ROWS // LANES


def _build_sc_gather():
    mesh = plsc.VectorSubcoreMesh(core_axis_name="c", subcore_axis_name="s")

    @functools.partial(
        pl.kernel,
        mesh=mesh,
        compiler_params=pltpu.CompilerParams(use_tc_tiling_on_sc=False,
                                             needs_layout_passes=False),
        out_type=jax.ShapeDtypeStruct((BATCH, NUM_FIELDS * EMB_DIM),
                                      jnp.float32),
        scratch_types=[
            pltpu.VMEM((P_ROWS, NUM_FIELDS), jnp.int32),
            pltpu.VMEM((NUM_FIELDS, P_ROWS), jnp.int32),
            pltpu.VMEM((NUM_FIELDS, P_ROWS, EMB_DIM), jnp.float32),
            pltpu.SemaphoreType.DMA,
            pltpu.SemaphoreType.DMA,
        ],
    )
    def k(x_hbm, tab_hbm, out_hbm, xblk_v, idx_v, rows_v, gsem, wsem):
        wid = lax.axis_index("s") * NC + lax.axis_index("c")

        def one_pass(p, carry):
            base = wid * ROWS_PER_W + p * P_ROWS
            pltpu.sync_copy(x_hbm.at[pl.ds(base, P_ROWS)], xblk_v)

            gathers = []
            for f in range(NUM_FIELDS):
                for kk in range(N_VEC):
                    rows = lax.iota(jnp.int32, LANES) + (kk * LANES)
                    cols = jnp.full((LANES,), f, dtype=jnp.int32)
                    vals = plsc.load_gather(xblk_v, [rows, cols])
                    idx_v[f, pl.ds(kk * LANES, LANES)] = vals
                gathers.append(pltpu.async_copy(
                    tab_hbm.at[f].at[idx_v.at[f]],
                    rows_v.at[f],
                    gsem))
            for g in gathers:
                g.wait()

            writes = []
            for f in range(NUM_FIELDS):
                writes.append(pltpu.async_copy(
                    rows_v.at[f],
                    out_hbm.at[pl.ds(base, P_ROWS),
                               pl.ds(f * EMB_DIM, EMB_DIM)],
                    wsem))
            for w in writes:
                w.wait()
            return carry

        lax.fori_loop(0, N_PASS, one_pass, 0)

    return k


def kernel(x, tables):
    return _build_sc_gather()(x, tables)


# speedup vs baseline: 1.9681x; 1.7210x over previous
import functools

import jax
import jax.numpy as jnp
from jax import lax
from jax.experimental import pallas as pl
from jax.experimental.pallas import tpu as pltpu
from jax.experimental.pallas import tpu_sc as plsc

NUM_FIELDS = 26
VOCAB = 100000
EMB_DIM = 16
BATCH = 16384

NC = 2
NS = 16
NW = NC * NS
ROWS_PER_W = BATCH // NW
P_ROWS = 128
N_PASS = ROWS_PER_W // P_ROWS


def _build_sc_gather():
    mesh = plsc.VectorSubcoreMesh(core_axis_name="c", subcore_axis_name="s")

    @functools.partial(
        pl.kernel,
        mesh=mesh,
        compiler_params=pltpu.CompilerParams(use_tc_tiling_on_sc=False,
                                             needs_layout_passes=False),
        out_type=jax.ShapeDtypeStruct((NUM_FIELDS * EMB_DIM, BATCH),
                                      jnp.float32),
        scratch_types=[
            pltpu.VMEM((P_ROWS,), jnp.int32),
            pltpu.VMEM((EMB_DIM, P_ROWS), jnp.float32),
            pltpu.SemaphoreType.DMA,
            pltpu.SemaphoreType.DMA,
        ],
    )
    def k(xt_hbm, tabt_hbm, out_hbm, idx_v, blk_v, gsem, wsem):
        wid = lax.axis_index("s") * NC + lax.axis_index("c")

        def one_pass(p, carry):
            b0 = wid * ROWS_PER_W + p * P_ROWS

            def one_field(f, c2):
                pltpu.sync_copy(xt_hbm.at[f, pl.ds(b0, P_ROWS)], idx_v)
                gathers = []
                for c in range(EMB_DIM):
                    gathers.append(pltpu.async_copy(
                        tabt_hbm.at[f, c].at[idx_v],
                        blk_v.at[c],
                        gsem))
                for g in gathers:
                    g.wait()
                pltpu.async_copy(
                    blk_v,
                    out_hbm.at[pl.ds(f * EMB_DIM, EMB_DIM),
                               pl.ds(b0, P_ROWS)],
                    wsem).wait()
                return c2

            lax.fori_loop(0, NUM_FIELDS, one_field, 0)
            return carry

        lax.fori_loop(0, N_PASS, one_pass, 0)

    return k


def kernel(x, tables):
    xt = x.T
    tabt = jnp.swapaxes(tables, 1, 2)
    out_t = _build_sc_gather()(xt, tabt)
    return out_t.T


# speedup vs baseline: 2.1702x; 1.1027x over previous
import functools

import jax
import jax.numpy as jnp
from jax import lax
from jax.experimental import pallas as pl
from jax.experimental.pallas import tpu as pltpu
from jax.experimental.pallas import tpu_sc as plsc

NUM_FIELDS = 26
VOCAB = 100000
EMB_DIM = 16
BATCH = 16384

NC = 2
NS = 16
NW = NC * NS
ROWS_PER_W = BATCH // NW
P_ROWS = 128
N_PASS = ROWS_PER_W // P_ROWS


def _build_sc_gather():
    mesh = plsc.VectorSubcoreMesh(core_axis_name="c", subcore_axis_name="s")

    @functools.partial(
        pl.kernel,
        mesh=mesh,
        compiler_params=pltpu.CompilerParams(use_tc_tiling_on_sc=False,
                                             needs_layout_passes=False),
        out_type=jax.ShapeDtypeStruct((NUM_FIELDS * EMB_DIM, BATCH),
                                      jnp.float32),
        scratch_types=[
            pltpu.VMEM((NUM_FIELDS, P_ROWS), jnp.int32),
            pltpu.VMEM((2, EMB_DIM, P_ROWS), jnp.float32),
            pltpu.SemaphoreType.DMA,
            pltpu.SemaphoreType.DMA,
        ],
    )
    def k(xt_hbm, tabt_hbm, out_hbm, idx_v, blk_v, gsem, wsem):
        wid = lax.axis_index("s") * NC + lax.axis_index("c")

        def one_pass(p, carry):
            b0 = wid * ROWS_PER_W + p * P_ROWS
            pltpu.sync_copy(xt_hbm.at[:, pl.ds(b0, P_ROWS)], idx_v)

            def one_field(f, c2):
                slot = lax.rem(f, 2)
                u = p * NUM_FIELDS + f

                @pl.when(u >= 2)
                def _():
                    pltpu.make_async_copy(
                        blk_v.at[slot],
                        out_hbm.at[pl.ds(0, EMB_DIM), pl.ds(0, P_ROWS)],
                        wsem).wait()

                gathers = []
                for c in range(EMB_DIM):
                    gathers.append(pltpu.async_copy(
                        tabt_hbm.at[f, c].at[idx_v.at[f]],
                        blk_v.at[slot, c],
                        gsem))
                for g in gathers:
                    g.wait()

                pltpu.async_copy(
                    blk_v.at[slot],
                    out_hbm.at[pl.ds(f * EMB_DIM, EMB_DIM),
                               pl.ds(b0, P_ROWS)],
                    wsem)
                return c2

            lax.fori_loop(0, NUM_FIELDS, one_field, 0)
            return carry

        lax.fori_loop(0, N_PASS, one_pass, 0)

        for slot in range(2):
            pltpu.make_async_copy(
                blk_v.at[slot],
                out_hbm.at[pl.ds(0, EMB_DIM), pl.ds(0, P_ROWS)],
                wsem).wait()

    return k


def kernel(x, tables):
    xt = x.T
    tabt = jnp.swapaxes(tables, 1, 2)
    out_t = _build_sc_gather()(xt, tabt)
    return out_t.T
